# baseline (speedup 1.0000x reference)
import functools

import jax
import jax.numpy as jnp
from jax import lax
from jax.experimental import pallas as pl
from jax.experimental.pallas import tpu as pltpu
from jax.experimental.pallas import tpu_sc as plsc

N = 10000
E = 320000
F_IN = 128
C = 40
D = 40
NC = 2
NS = 16
NW = NC * NS
EPW = E // NW
CH = 125
NCH = EPW // CH
NPT = N // NS

_mesh = plsc.VectorSubcoreMesh(core_axis_name="c", subcore_axis_name="s")


def _deg_body(col_hbm, degp_hbm, colv, acc):
    cid = lax.axis_index("c")
    sid = lax.axis_index("s")
    wid = cid * NS + sid
    zeros16 = jnp.zeros((16,), jnp.float32)

    def zbody(i, _):
        acc[pl.ds(i * 16, 16)] = zeros16
        return ()
    lax.fori_loop(0, N // 16, zbody, (), unroll=8)

    pltpu.sync_copy(col_hbm.at[pl.ds(wid * EPW, EPW)], colv)
    ones16 = jnp.ones((16,), jnp.float32)

    def body(i, _):
        idx = colv[pl.ds(i * 16, 16)]
        plsc.addupdate_scatter(acc, [idx], ones16)
        return ()
    lax.fori_loop(0, EPW // 16, body, (), unroll=4)

    pltpu.sync_copy(acc, degp_hbm.at[pl.ds(wid * N, N)])


_deg_call = functools.partial(
    pl.kernel,
    out_type=jax.ShapeDtypeStruct((NW * N,), jnp.float32),
    mesh=_mesh,
    scratch_types=[
        pltpu.VMEM((EPW,), jnp.int32),
        pltpu.VMEM((N,), jnp.float32),
    ],
    compiler_params=pltpu.CompilerParams(needs_layout_passes=False, use_tc_tiling_on_sc=False),
)(_deg_body)


NSLOT = 8


def _hop_body(z_hbm, row_hbm, col_hbm, zeros_hbm, s_hbm,
              rowi, coli, bufs, acc_sh, sgs, sss):
    cid = lax.axis_index("c")
    sid = lax.axis_index("s")
    wid = cid * NS + sid

    @pl.when(sid < NS - 1)
    def _():
        st = pl.multiple_of(sid * 640, 8)
        pltpu.sync_copy(zeros_hbm.at[pl.ds(st, 640)], acc_sh.at[pl.ds(st, 640)])

    @pl.when(sid == NS - 1)
    def _():
        pltpu.sync_copy(zeros_hbm.at[pl.ds(9600, 400)],
                        acc_sh.at[pl.ds(9600, 400)])
    pltpu.sync_copy(row_hbm.at[pl.ds(wid * NCH, NCH)], rowi)
    pltpu.sync_copy(col_hbm.at[pl.ds(wid * NCH, NCH)], coli)
    plsc.subcore_barrier()

    for b in range(NSLOT):
        pltpu.async_copy(z_hbm.at[rowi.at[b]], bufs[b], sgs[b])

    def t_body(t, _):
        j = t * NSLOT
        for b in range(NSLOT):
            pltpu.make_async_copy(z_hbm.at[rowi.at[j + b]], bufs[b], sgs[b]).wait()
            pltpu.async_copy(bufs[b], acc_sh.at[coli.at[j + b]], sss[b], add=True)
        for b in range(NSLOT):
            pltpu.make_async_copy(bufs[b], acc_sh.at[coli.at[j + b]], sss[b]).wait()

            @pl.when(t < NCH // NSLOT - 1)
            def _():
                pltpu.async_copy(z_hbm.at[rowi.at[j + NSLOT + b]], bufs[b], sgs[b])
        return ()
    lax.fori_loop(0, NCH // NSLOT, t_body, ())

    plsc.subcore_barrier()

    @pl.when(sid < NS - 1)
    def _():
        st = pl.multiple_of(sid * 640, 8)
        pltpu.sync_copy(acc_sh.at[pl.ds(st, 640)],
                        s_hbm.at[cid, pl.ds(st, 640)])

    @pl.when(sid == NS - 1)
    def _():
        pltpu.sync_copy(acc_sh.at[pl.ds(9600, 400)],
                        s_hbm.at[cid, pl.ds(9600, 400)])


_hop_call = functools.partial(
    pl.kernel,
    out_type=jax.ShapeDtypeStruct((NC, N, D), jnp.float32),
    mesh=_mesh,
    scratch_types=[
        pltpu.VMEM((NCH, CH), jnp.int32),
        pltpu.VMEM((NCH, CH), jnp.int32),
        [pltpu.VMEM((CH, D), jnp.float32) for _ in range(NSLOT)],
        pltpu.VMEM_SHARED((N, D), jnp.float32),
        [pltpu.SemaphoreType.DMA for _ in range(NSLOT)],
        [pltpu.SemaphoreType.DMA for _ in range(NSLOT)],
    ],
    compiler_params=pltpu.CompilerParams(needs_layout_passes=False, use_tc_tiling_on_sc=False),
)(_hop_body)


BN = 1000


def _dinv(degp_blk):
    deg = jnp.sum(degp_blk, axis=1) + 1.0
    return lax.rsqrt(deg)


def _z0_body(degp_ref, x_ref, w_ref, z0_ref):
    dinv = _dinv(degp_ref[...])
    y = jnp.dot(x_ref[...], w_ref[...].T, preferred_element_type=jnp.float32)
    z0_ref[...] = dinv[:, None] * y


_z0_call = pl.pallas_call(
    _z0_body,
    grid=(N // BN,),
    in_specs=[
        pl.BlockSpec((BN, NW), lambda i: (i, 0)),
        pl.BlockSpec((BN, F_IN), lambda i: (i, 0)),
        pl.BlockSpec((D, F_IN), lambda i: (0, 0)),
    ],
    out_specs=pl.BlockSpec((BN, D), lambda i: (i, 0)),
    out_shape=jax.ShapeDtypeStruct((N, D), jnp.float32),
)


def _scale_body(degp_ref, s_ref, z_ref, o_ref):
    deg = jnp.sum(degp_ref[...], axis=1) + 1.0
    o_ref[...] = (1.0 / deg)[:, None] * (s_ref[0] + s_ref[1] + z_ref[...])


_scale_call = pl.pallas_call(
    _scale_body,
    grid=(N // BN,),
    in_specs=[
        pl.BlockSpec((BN, NW), lambda i: (i, 0)),
        pl.BlockSpec((NC, BN, D), lambda i: (0, i, 0)),
        pl.BlockSpec((BN, D), lambda i: (i, 0)),
    ],
    out_specs=pl.BlockSpec((BN, D), lambda i: (i, 0)),
    out_shape=jax.ShapeDtypeStruct((N, D), jnp.float32),
)


def _final_body(degp_ref, s_ref, z_ref, b_ref, o_ref):
    dinv = _dinv(degp_ref[...])
    h = dinv[:, None] * (s_ref[0] + s_ref[1] + z_ref[...])
    logits = h[:, :C] + b_ref[...]
    m = jnp.max(logits, axis=1, keepdims=True)
    lse = jnp.log(jnp.sum(jnp.exp(logits - m), axis=1, keepdims=True))
    o_ref[...] = logits - m - lse


_final_call = pl.pallas_call(
    _final_body,
    grid=(N // BN,),
    in_specs=[
        pl.BlockSpec((BN, NW), lambda i: (i, 0)),
        pl.BlockSpec((NC, BN, D), lambda i: (0, i, 0)),
        pl.BlockSpec((BN, D), lambda i: (i, 0)),
        pl.BlockSpec((1, C), lambda i: (0, 0)),
    ],
    out_specs=pl.BlockSpec((BN, C), lambda i: (i, 0)),
    out_shape=jax.ShapeDtypeStruct((N, C), jnp.float32),
)


def kernel(x, edge_index, W, b):
    row = edge_index[0].astype(jnp.int32)
    col = edge_index[1].astype(jnp.int32)
    row2d = row.reshape(E // CH, CH)
    col2d = col.reshape(E // CH, CH)
    zeros_nd = jnp.zeros((N, D), jnp.float32)

    degp = _deg_call(col).reshape(NW, N).T
    z0 = _z0_call(degp, x, W)
    s1 = _hop_call(z0, row2d, col2d, zeros_nd)
    z1 = _scale_call(degp, s1, z0)
    s2 = _hop_call(z1, row2d, col2d, zeros_nd)
    return _final_call(degp, s2, z1, b.reshape(1, C))
